# baseline (speedup 1.0000x reference)
import functools

import jax
import jax.numpy as jnp
from jax import lax
from jax.experimental import pallas as pl
from jax.experimental.pallas import tpu as pltpu
from jax.experimental.pallas import tpu_sc as plsc

DP = 128
LW = 128
LANES = 16


def kernel(tokens, W_E):
    B, P = tokens.shape
    V, D = W_E.shape
    N = B * P

    info = plsc.get_sparse_core_info()
    NC, NS = info.num_cores, info.num_subcores
    NW = NC * NS
    assert B == NW * LW and DP == 2 * D and (V * D) % DP == 0 and P % 2 == 0
    TPW = LW * P
    NG = LW // LANES

    tok_flat = tokens.reshape(N).astype(jnp.int32)
    W2 = jnp.reshape(W_E, (V * D // DP, DP))

    mesh = plsc.VectorSubcoreMesh(core_axis_name="c", subcore_axis_name="s")

    @functools.partial(
        pl.kernel,
        mesh=mesh,
        compiler_params=pltpu.CompilerParams(
            use_tc_tiling_on_sc=True, needs_layout_passes=False
        ),
        out_type=jax.ShapeDtypeStruct((P, D, B), jnp.float32),
        scratch_types=[
            pltpu.VMEM((TPW,), jnp.int32),
            pltpu.VMEM((P, LW), jnp.int32),
            pltpu.VMEM((2, LW), jnp.int32),
            pltpu.VMEM((2, LW, DP), jnp.float32),
            pltpu.VMEM((2, D, LW), jnp.float32),
            pltpu.SemaphoreType.DMA,
            pltpu.SemaphoreType.DMA,
            pltpu.SemaphoreType.DMA,
            pltpu.SemaphoreType.DMA,
        ],
    )
    def emb(tok_hbm, table_hbm, out_hbm, idx_v, idx_t, idx_g, rows_v,
            trans_v, g0, g1, o0, o1):
        wid = lax.axis_index("s") * NC + lax.axis_index("c")
        b0 = wid * LW
        pltpu.sync_copy(tok_hbm.at[pl.ds(b0 * P, TPW)], idx_v)

        iota = lax.iota(jnp.int32, LANES)
        iota_p = iota * P

        def tbody(p, carry):
            for g in range(NG):
                vec = iota_p + (g * LANES * P + p)
                idx_t[p, pl.ds(g * LANES, LANES)] = plsc.load_gather(
                    idx_v, [vec]
                )
            return carry

        lax.fori_loop(0, P, tbody, 0)

        def prep_fire(p, s, gsem):
            for g in range(NG):
                v = idx_t[p, pl.ds(g * LANES, LANES)]
                idx_g[s, pl.ds(g * LANES, LANES)] = (
                    lax.shift_right_logical(v, 1)
                )
            pltpu.async_copy(table_hbm.at[idx_g.at[s]], rows_v.at[s], gsem)

        def drain_gather(gsem):
            pltpu.make_async_copy(
                table_hbm.at[pl.ds(0, LW)], rows_v.at[0], gsem
            ).wait()

        def transpose(p, s):
            src = rows_v.at[s]
            for g in range(NG):
                tok = idx_t[p, pl.ds(g * LANES, LANES)]
                col_base = (tok & 1) * D
                row = iota + (g * LANES)
                for f in range(D):
                    v = plsc.load_gather(src, [row, col_base + f])
                    trans_v[s, f, pl.ds(g * LANES, LANES)] = v

        def start_store(p, s, osem):
            pltpu.async_copy(
                trans_v.at[s], out_hbm.at[p, :, pl.ds(b0, LW)], osem
            )

        def drain_store(osem):
            pltpu.make_async_copy(
                trans_v.at[0], out_hbm.at[0, :, pl.ds(0, LW)], osem
            ).wait()

        prep_fire(0, 0, g0)

        def body(rp, carry):
            p0 = 2 * rp
            drain_gather(g0)
            prep_fire(p0 + 1, 1, g1)
            @pl.when(rp >= 1)
            def _():
                drain_store(o0)
            transpose(p0, 0)
            start_store(p0, 0, o0)
            drain_gather(g1)
            @pl.when(rp < P // 2 - 1)
            def _():
                prep_fire(p0 + 2, 0, g0)
            @pl.when(rp >= 1)
            def _():
                drain_store(o1)
            transpose(p0 + 1, 1)
            start_store(p0 + 1, 1, o1)
            return carry

        lax.fori_loop(0, P // 2, body, 0)
        drain_store(o0)
        drain_store(o1)

    out_k = emb(tok_flat, W2)
    return jnp.transpose(out_k, (2, 0, 1))


# speedup vs baseline: 1.5197x; 1.5197x over previous
import functools

import jax
import jax.numpy as jnp
from jax import lax
from jax.experimental import pallas as pl
from jax.experimental.pallas import tpu as pltpu
from jax.experimental.pallas import tpu_sc as plsc

DP = 128
LW = 128
LANES = 16


def kernel(tokens, W_E):
    B, P = tokens.shape
    V, D = W_E.shape
    N = B * P

    info = plsc.get_sparse_core_info()
    NC, NS = info.num_cores, info.num_subcores
    NW = NC * NS
    assert B == NW * LW and DP == 2 * D and (V * D) % DP == 0 and P % 2 == 0
    TPW = LW * P
    NG = LW // LANES

    tok_flat = tokens.reshape(N).astype(jnp.int32)
    W2 = jnp.reshape(W_E, (V * D // DP, DP))

    mesh = plsc.VectorSubcoreMesh(core_axis_name="c", subcore_axis_name="s")

    @functools.partial(
        pl.kernel,
        mesh=mesh,
        compiler_params=pltpu.CompilerParams(
            use_tc_tiling_on_sc=True, needs_layout_passes=False
        ),
        out_type=jax.ShapeDtypeStruct((P, D, B), jnp.float32),
        scratch_types=[
            pltpu.VMEM((TPW,), jnp.int32),
            pltpu.VMEM((P, LW), jnp.int32),
            pltpu.VMEM((2, LW), jnp.int32),
            pltpu.VMEM((2, LW, DP), jnp.float32),
            pltpu.VMEM((2, D, LW), jnp.float32),
            pltpu.SemaphoreType.DMA,
            pltpu.SemaphoreType.DMA,
            pltpu.SemaphoreType.DMA,
            pltpu.SemaphoreType.DMA,
        ],
    )
    def emb(tok_hbm, table_hbm, out_hbm, idx_v, idx_t, idx_g, rows_v,
            trans_v, g0, g1, o0, o1):
        wid = lax.axis_index("s") * NC + lax.axis_index("c")
        b0 = wid * LW
        pltpu.sync_copy(tok_hbm.at[pl.ds(b0 * P, TPW)], idx_v)

        iota = lax.iota(jnp.int32, LANES)
        iota_p = iota * P

        def tbody(p, carry):
            for g in range(NG):
                vec = iota_p + (g * LANES * P + p)
                idx_t[p, pl.ds(g * LANES, LANES)] = plsc.load_gather(
                    idx_v, [vec]
                )
            return carry

        lax.fori_loop(0, P, tbody, 0)

        def prep_fire(p, s, gsem):
            for g in range(NG):
                v = idx_t[p, pl.ds(g * LANES, LANES)]
                idx_g[s, pl.ds(g * LANES, LANES)] = (
                    lax.shift_right_logical(v, 1)
                )
            pltpu.async_copy(table_hbm.at[idx_g.at[s]], rows_v.at[s], gsem)

        def drain_gather(gsem):
            pltpu.make_async_copy(
                table_hbm.at[pl.ds(0, LW)], rows_v.at[0], gsem
            ).wait()

        def transpose(p, s):
            src = rows_v.at[s]
            for g in range(NG):
                tok = idx_t[p, pl.ds(g * LANES, LANES)]
                col_base = (tok & 1) * D
                row = iota + (g * LANES)

                @plsc.parallel_loop(0, D, unroll=8)
                def _(f):
                    v = plsc.load_gather(src, [row, col_base + f])
                    trans_v[s, f, pl.ds(g * LANES, LANES)] = v

        def start_store(p, s, osem):
            pltpu.async_copy(
                trans_v.at[s], out_hbm.at[p, :, pl.ds(b0, LW)], osem
            )

        def drain_store(osem):
            pltpu.make_async_copy(
                trans_v.at[0], out_hbm.at[0, :, pl.ds(0, LW)], osem
            ).wait()

        prep_fire(0, 0, g0)

        def body(rp, carry):
            p0 = 2 * rp
            drain_gather(g0)
            prep_fire(p0 + 1, 1, g1)
            @pl.when(rp >= 1)
            def _():
                drain_store(o0)
            transpose(p0, 0)
            start_store(p0, 0, o0)
            drain_gather(g1)
            @pl.when(rp < P // 2 - 1)
            def _():
                prep_fire(p0 + 2, 0, g0)
            @pl.when(rp >= 1)
            def _():
                drain_store(o1)
            transpose(p0 + 1, 1)
            start_store(p0 + 1, 1, o1)
            return carry

        lax.fori_loop(0, P // 2, body, 0)
        drain_store(o0)
        drain_store(o1)

    out_k = emb(tok_flat, W2)
    return jnp.transpose(out_k, (2, 0, 1))


# speedup vs baseline: 1.5454x; 1.0169x over previous
import functools

import jax
import jax.numpy as jnp
from jax import lax
from jax.experimental import pallas as pl
from jax.experimental.pallas import tpu as pltpu
from jax.experimental.pallas import tpu_sc as plsc

DP = 128
LW = 128
LANES = 16


def kernel(tokens, W_E):
    B, P = tokens.shape
    V, D = W_E.shape
    N = B * P

    info = plsc.get_sparse_core_info()
    NC, NS = info.num_cores, info.num_subcores
    NW = NC * NS
    assert B == NW * LW and DP == 2 * D and (V * D) % DP == 0 and P % 2 == 0
    TPW = LW * P
    NG = LW // LANES

    tok_flat = tokens.reshape(N).astype(jnp.int32)
    W2 = jnp.reshape(W_E, (V * D // DP, DP))

    mesh = plsc.VectorSubcoreMesh(core_axis_name="c", subcore_axis_name="s")

    @functools.partial(
        pl.kernel,
        mesh=mesh,
        compiler_params=pltpu.CompilerParams(
            use_tc_tiling_on_sc=True, needs_layout_passes=False
        ),
        out_type=jax.ShapeDtypeStruct((P, D, B), jnp.float32),
        scratch_types=[
            pltpu.VMEM((TPW,), jnp.int32),
            pltpu.VMEM((P, LW), jnp.int32),
            pltpu.VMEM((2, LW), jnp.int32),
            pltpu.VMEM((2, LW, DP), jnp.float32),
            pltpu.VMEM((2, D, LW), jnp.float32),
            pltpu.SemaphoreType.DMA,
            pltpu.SemaphoreType.DMA,
            pltpu.SemaphoreType.DMA,
            pltpu.SemaphoreType.DMA,
        ],
    )
    def emb(tok_hbm, table_hbm, out_hbm, idx_v, idx_t, idx_g, rows_v,
            trans_v, g0, g1, o0, o1):
        wid = lax.axis_index("s") * NC + lax.axis_index("c")
        b0 = wid * LW
        pltpu.sync_copy(tok_hbm.at[pl.ds(b0 * P, TPW)], idx_v)

        iota = lax.iota(jnp.int32, LANES)
        iota_p = iota * P

        @plsc.parallel_loop(0, P, unroll=4)
        def _(p):
            for g in range(NG):
                vec = iota_p + (g * LANES * P + p)
                idx_t[p, pl.ds(g * LANES, LANES)] = plsc.load_gather(
                    idx_v, [vec]
                )

        def prep_fire(p, s, gsem):
            for g in range(NG):
                v = idx_t[p, pl.ds(g * LANES, LANES)]
                idx_g[s, pl.ds(g * LANES, LANES)] = (
                    lax.shift_right_logical(v, 1)
                )
            pltpu.async_copy(table_hbm.at[idx_g.at[s]], rows_v.at[s], gsem)

        def drain_gather(gsem):
            pltpu.make_async_copy(
                table_hbm.at[pl.ds(0, LW)], rows_v.at[0], gsem
            ).wait()

        def transpose(p, s):
            src = rows_v.at[s]
            col_bases = []
            rows = []
            for g in range(NG):
                tok = idx_t[p, pl.ds(g * LANES, LANES)]
                col_bases.append((tok & 1) * D)
                rows.append(iota + (g * LANES))

            @plsc.parallel_loop(0, D, unroll=4)
            def _(f):
                for g in range(NG):
                    v = plsc.load_gather(src, [rows[g], col_bases[g] + f])
                    trans_v[s, f, pl.ds(g * LANES, LANES)] = v

        def start_store(p, s, osem):
            pltpu.async_copy(
                trans_v.at[s], out_hbm.at[p, :, pl.ds(b0, LW)], osem
            )

        def drain_store(osem):
            pltpu.make_async_copy(
                trans_v.at[0], out_hbm.at[0, :, pl.ds(0, LW)], osem
            ).wait()

        prep_fire(0, 0, g0)

        def body(rp, carry):
            p0 = 2 * rp
            drain_gather(g0)
            prep_fire(p0 + 1, 1, g1)
            @pl.when(rp >= 1)
            def _():
                drain_store(o0)
            transpose(p0, 0)
            start_store(p0, 0, o0)
            drain_gather(g1)
            @pl.when(rp < P // 2 - 1)
            def _():
                prep_fire(p0 + 2, 0, g0)
            @pl.when(rp >= 1)
            def _():
                drain_store(o1)
            transpose(p0 + 1, 1)
            start_store(p0 + 1, 1, o1)
            return carry

        lax.fori_loop(0, P // 2, body, 0)
        drain_store(o0)
        drain_store(o1)

    out_k = emb(tok_flat, W2)
    return jnp.transpose(out_k, (2, 0, 1))


# speedup vs baseline: 2.1023x; 1.3604x over previous
import functools

import jax
import jax.numpy as jnp
from jax import lax
from jax.experimental import pallas as pl
from jax.experimental.pallas import tpu as pltpu
from jax.experimental.pallas import tpu_sc as plsc

DP = 128
LW = 128
LANES = 16


def kernel(tokens, W_E):
    B, P = tokens.shape
    V, D = W_E.shape
    N = B * P

    info = plsc.get_sparse_core_info()
    NC, NS = info.num_cores, info.num_subcores
    NW = NC * NS
    assert B == NW * LW and DP == 2 * D and (V * D) % DP == 0 and P % 2 == 0
    TPW = LW * P
    NG = LW // LANES

    tok_flat = tokens.reshape(N).astype(jnp.int32)
    W2 = jnp.reshape(W_E, (V * D // DP, DP))

    mesh = plsc.VectorSubcoreMesh(core_axis_name="c", subcore_axis_name="s")

    @functools.partial(
        pl.kernel,
        mesh=mesh,
        compiler_params=pltpu.CompilerParams(
            use_tc_tiling_on_sc=True, needs_layout_passes=False
        ),
        out_type=jax.ShapeDtypeStruct((P, D, B), jnp.float32),
        scratch_types=[
            pltpu.VMEM((TPW,), jnp.int32),
            pltpu.VMEM((P, LW), jnp.int32),
            pltpu.VMEM((2, LW), jnp.int32),
            pltpu.VMEM((2, LW, DP), jnp.float32),
            pltpu.VMEM((2, D, LW), jnp.float32),
            pltpu.SemaphoreType.DMA,
            pltpu.SemaphoreType.DMA,
            pltpu.SemaphoreType.DMA,
            pltpu.SemaphoreType.DMA,
        ],
    )
    def emb(tok_hbm, table_hbm, out_hbm, idx_v, idx_t, idx_g, rows_v,
            trans_v, g0, g1, o0, o1):
        wid = lax.axis_index("s") * NC + lax.axis_index("c")
        b0 = wid * LW
        pltpu.sync_copy(tok_hbm.at[pl.ds(b0 * P, TPW)], idx_v)

        iota = lax.iota(jnp.int32, LANES)
        iota_p = iota * P

        @plsc.parallel_loop(0, P, unroll=4)
        def _(p):
            for g in range(NG):
                vec = iota_p + (g * LANES * P + p)
                idx_t[p, pl.ds(g * LANES, LANES)] = plsc.load_gather(
                    idx_v, [vec]
                )

        def prep_fire(p, s, gsem):
            for g in range(NG):
                v = idx_t[p, pl.ds(g * LANES, LANES)]
                idx_g[s, pl.ds(g * LANES, LANES)] = (
                    lax.shift_right_logical(v, 1)
                )
            pltpu.async_copy(table_hbm.at[idx_g.at[s]], rows_v.at[s], gsem)

        def drain_gather(gsem):
            pltpu.make_async_copy(
                table_hbm.at[pl.ds(0, LW)], rows_v.at[0], gsem
            ).wait()

        def transpose(p, s):
            src = rows_v.at[s]
            col_bases = []
            rows = []
            for g in range(NG):
                tok = idx_t[p, pl.ds(g * LANES, LANES)]
                col_bases.append((tok & 1) * D)
                rows.append(iota + (g * LANES))

            dst = trans_v.at[s]

            @plsc.parallel_loop(0, LANES, unroll=2)
            def _(j):
                r = (iota + j) & (LANES - 1)
                for fb in range(D // LANES):
                    r_fb = r + fb * LANES
                    for g in range(NG):
                        v = plsc.load_gather(
                            src, [rows[g], col_bases[g] + r_fb]
                        )
                        plsc.store_scatter(dst, [r_fb, rows[g]], v)

        def start_store(p, s, osem):
            pltpu.async_copy(
                trans_v.at[s], out_hbm.at[p, :, pl.ds(b0, LW)], osem
            )

        def drain_store(osem):
            pltpu.make_async_copy(
                trans_v.at[0], out_hbm.at[0, :, pl.ds(0, LW)], osem
            ).wait()

        prep_fire(0, 0, g0)

        def body(rp, carry):
            p0 = 2 * rp
            drain_gather(g0)
            prep_fire(p0 + 1, 1, g1)
            @pl.when(rp >= 1)
            def _():
                drain_store(o0)
            transpose(p0, 0)
            start_store(p0, 0, o0)
            drain_gather(g1)
            @pl.when(rp < P // 2 - 1)
            def _():
                prep_fire(p0 + 2, 0, g0)
            @pl.when(rp >= 1)
            def _():
                drain_store(o1)
            transpose(p0 + 1, 1)
            start_store(p0 + 1, 1, o1)
            return carry

        lax.fori_loop(0, P // 2, body, 0)
        drain_store(o0)
        drain_store(o1)

    out_k = emb(tok_flat, W2)
    return jnp.transpose(out_k, (2, 0, 1))


# speedup vs baseline: 2.3343x; 1.1103x over previous
import functools

import jax
import jax.numpy as jnp
from jax import lax
from jax.experimental import pallas as pl
from jax.experimental.pallas import tpu as pltpu
from jax.experimental.pallas import tpu_sc as plsc

DP = 128
LW = 128
LANES = 16
NB = 4


def kernel(tokens, W_E):
    B, P = tokens.shape
    V, D = W_E.shape
    N = B * P

    info = plsc.get_sparse_core_info()
    NC, NS = info.num_cores, info.num_subcores
    NW = NC * NS
    assert B == NW * LW and DP == 2 * D and (V * D) % DP == 0
    assert P % NB == 0 and D % LANES == 0 and LW % (2 * LANES) == 0
    TPW = LW * P
    NG = LW // LANES
    HSTG = TPW // 2

    tok_flat = tokens.reshape(N).astype(jnp.int32)
    W2 = jnp.reshape(W_E, (V * D // DP, DP))

    mesh = plsc.VectorSubcoreMesh(core_axis_name="c", subcore_axis_name="s")

    @functools.partial(
        pl.kernel,
        mesh=mesh,
        compiler_params=pltpu.CompilerParams(
            use_tc_tiling_on_sc=True, needs_layout_passes=False
        ),
        out_type=jax.ShapeDtypeStruct((P, D, B), jnp.float32),
        scratch_types=[
            pltpu.VMEM((HSTG,), jnp.int32),
            pltpu.VMEM((P, LW), jnp.int32),
            pltpu.VMEM((NB, LW), jnp.int32),
            pltpu.VMEM((NB, LW, DP), jnp.float32),
            pltpu.VMEM((2, D, LW), jnp.float32),
            pltpu.SemaphoreType.DMA,
            pltpu.SemaphoreType.DMA,
            pltpu.SemaphoreType.DMA,
            pltpu.SemaphoreType.DMA,
            pltpu.SemaphoreType.DMA,
            pltpu.SemaphoreType.DMA,
        ],
    )
    def emb(tok_hbm, table_hbm, out_hbm, idx_v, idx_t, idx_g, rows_v,
            trans_v, g0, g1, g2, g3, o0, o1):
        gsem = [g0, g1, g2, g3]
        osem = [o0, o1]
        wid = lax.axis_index("s") * NC + lax.axis_index("c")
        b0 = wid * LW

        iota = lax.iota(jnp.int32, LANES)
        iota_p = iota * P

        for h in range(2):
            pltpu.sync_copy(
                tok_hbm.at[pl.ds(b0 * P + h * HSTG, HSTG)], idx_v
            )

            @plsc.parallel_loop(0, P, unroll=4)
            def _(p):
                for g in range(NG // 2):
                    vec = iota_p + (g * LANES * P + p)
                    lbase = (h * (NG // 2) + g) * LANES
                    idx_t[p, pl.ds(lbase, LANES)] = plsc.load_gather(
                        idx_v, [vec]
                    )

        def prep_fire(p, k):
            for g in range(NG):
                v = idx_t[p, pl.ds(g * LANES, LANES)]
                idx_g[k, pl.ds(g * LANES, LANES)] = (
                    lax.shift_right_logical(v, 1)
                )
            pltpu.async_copy(
                table_hbm.at[idx_g.at[k]], rows_v.at[k], gsem[k]
            )

        def drain_gather(k):
            pltpu.make_async_copy(
                table_hbm.at[pl.ds(0, LW)], rows_v.at[0], gsem[k]
            ).wait()

        def transpose(p, k, t):
            src = rows_v.at[k]
            dst = trans_v.at[t]
            col_bases = []
            rows = []
            for g in range(NG):
                tok = idx_t[p, pl.ds(g * LANES, LANES)]
                col_bases.append((tok & 1) * D)
                rows.append(iota + (g * LANES))

            @plsc.parallel_loop(0, LANES, unroll=2)
            def _(j):
                r = (iota + j) & (LANES - 1)
                for fb in range(D // LANES):
                    r_fb = r + fb * LANES
                    for g in range(NG):
                        v = plsc.load_gather(
                            src, [rows[g], col_bases[g] + r_fb]
                        )
                        plsc.store_scatter(dst, [r_fb, rows[g]], v)

        def start_store(p, t):
            pltpu.async_copy(
                trans_v.at[t], out_hbm.at[p, :, pl.ds(b0, LW)], osem[t]
            )

        def drain_store(t):
            pltpu.make_async_copy(
                trans_v.at[0], out_hbm.at[0, :, pl.ds(0, LW)], osem[t]
            ).wait()

        prep_fire(0, 0)
        prep_fire(1, 1)

        def body(q, carry):
            p0 = NB * q
            for k in range(NB):
                p = p0 + k
                t = k % 2
                drain_gather(k)
                if k < 2:
                    prep_fire(p + 2, (k + 2) % NB)
                else:
                    @pl.when(q < P // NB - 1)
                    def _():
                        prep_fire(p + 2, (k + 2) % NB)
                @pl.when(q + (k // 2) >= 1)
                def _():
                    drain_store(t)
                transpose(p, k, t)
                start_store(p, t)
            return carry

        lax.fori_loop(0, P // NB, body, 0)
        drain_store(0)
        drain_store(1)

    out_k = emb(tok_flat, W2)
    return jnp.transpose(out_k, (2, 0, 1))


# speedup vs baseline: 2.3347x; 1.0002x over previous
import functools

import jax
import jax.numpy as jnp
from jax import lax
from jax.experimental import pallas as pl
from jax.experimental.pallas import tpu as pltpu
from jax.experimental.pallas import tpu_sc as plsc

DP = 128
LW = 128
LANES = 16
NB = 4


def kernel(tokens, W_E):
    B, P = tokens.shape
    V, D = W_E.shape
    N = B * P

    info = plsc.get_sparse_core_info()
    NC, NS = info.num_cores, info.num_subcores
    NW = NC * NS
    assert B == NW * LW and DP == 2 * D and (V * D) % DP == 0
    assert P % NB == 0 and D % LANES == 0 and LW % (2 * LANES) == 0
    TPW = LW * P
    NG = LW // LANES
    HSTG = TPW // 2

    tok_flat = tokens.reshape(N).astype(jnp.int32)
    W2 = jnp.reshape(W_E, (V * D // DP, DP))

    mesh = plsc.VectorSubcoreMesh(core_axis_name="c", subcore_axis_name="s")

    @functools.partial(
        pl.kernel,
        mesh=mesh,
        compiler_params=pltpu.CompilerParams(
            use_tc_tiling_on_sc=True, needs_layout_passes=False
        ),
        out_type=jax.ShapeDtypeStruct((P, D, B), jnp.float32),
        scratch_types=[
            pltpu.VMEM((HSTG,), jnp.int32),
            pltpu.VMEM((P, LW), jnp.int32),
            pltpu.VMEM((NB, LW), jnp.int32),
            pltpu.VMEM((NB, LW, DP), jnp.float32),
            pltpu.VMEM((2, D, LW), jnp.float32),
            pltpu.SemaphoreType.DMA,
            pltpu.SemaphoreType.DMA,
            pltpu.SemaphoreType.DMA,
            pltpu.SemaphoreType.DMA,
            pltpu.SemaphoreType.DMA,
            pltpu.SemaphoreType.DMA,
        ],
    )
    def emb(tok_hbm, table_hbm, out_hbm, idx_v, idx_t, idx_g, rows_v,
            trans_v, g0, g1, g2, g3, o0, o1):
        gsem = [g0, g1, g2, g3]
        osem = [o0, o1]
        wid = lax.axis_index("s") * NC + lax.axis_index("c")
        b0 = wid * LW

        iota = lax.iota(jnp.int32, LANES)
        iota_p = iota * P

        for h in range(2):
            pltpu.sync_copy(
                tok_hbm.at[pl.ds(b0 * P + h * HSTG, HSTG)], idx_v
            )

            @plsc.parallel_loop(0, P, unroll=4)
            def _(p):
                for g in range(NG // 2):
                    vec = iota_p + (g * LANES * P + p)
                    lbase = (h * (NG // 2) + g) * LANES
                    idx_t[p, pl.ds(lbase, LANES)] = plsc.load_gather(
                        idx_v, [vec]
                    )

        def prep_fire(p, k):
            for g in range(NG):
                v = idx_t[p, pl.ds(g * LANES, LANES)]
                idx_g[k, pl.ds(g * LANES, LANES)] = (
                    lax.shift_right_logical(v, 1)
                )
            pltpu.async_copy(
                table_hbm.at[idx_g.at[k]], rows_v.at[k], gsem[k]
            )

        def drain_gather(k):
            pltpu.make_async_copy(
                table_hbm.at[pl.ds(0, LW)], rows_v.at[0], gsem[k]
            ).wait()

        def transpose(p, k, t):
            src = rows_v.at[k]
            dst = trans_v.at[t]
            col_bases = []
            rows = []
            for g in range(NG):
                tok = idx_t[p, pl.ds(g * LANES, LANES)]
                col_bases.append((tok & 1) * D)
                rows.append(iota + (g * LANES))

            @plsc.parallel_loop(0, LANES, unroll=4)
            def _(j):
                r = (iota + j) & (LANES - 1)
                for fb in range(D // LANES):
                    r_fb = r + fb * LANES
                    for g in range(NG):
                        v = plsc.load_gather(
                            src, [rows[g], col_bases[g] + r_fb]
                        )
                        plsc.store_scatter(dst, [r_fb, rows[g]], v)

        def start_store(p, t):
            pltpu.async_copy(
                trans_v.at[t], out_hbm.at[p, :, pl.ds(b0, LW)], osem[t]
            )

        def drain_store(t):
            pltpu.make_async_copy(
                trans_v.at[0], out_hbm.at[0, :, pl.ds(0, LW)], osem[t]
            ).wait()

        prep_fire(0, 0)
        prep_fire(1, 1)

        def body(q, carry):
            p0 = NB * q
            for k in range(NB):
                p = p0 + k
                t = k % 2
                drain_gather(k)
                if k < 2:
                    prep_fire(p + 2, (k + 2) % NB)
                else:
                    @pl.when(q < P // NB - 1)
                    def _():
                        prep_fire(p + 2, (k + 2) % NB)
                @pl.when(q + (k // 2) >= 1)
                def _():
                    drain_store(t)
                transpose(p, k, t)
                start_store(p, t)
            return carry

        lax.fori_loop(0, P // NB, body, 0)
        drain_store(0)
        drain_store(1)

    out_k = emb(tok_flat, W2)
    return jnp.transpose(out_k, (2, 0, 1))
